# baseline (speedup 1.0000x reference)
import functools
import jax
import jax.numpy as jnp
from jax import lax
from jax.experimental import pallas as pl
from jax.experimental.pallas import tpu as pltpu
from jax.experimental.pallas import tpu_sc as plsc

N = 10000
E = 320000
D = 128
G = 64
CG = 32
NCORE = 2
NSUB = 16
NWORK = NCORE * NSUB
CH = 128
SUB = 64
NCHUNK_TOT = E // CH
CPT = 78
NEXTRA = NCHUNK_TOT - CPT * NWORK
IB = 26
NSUBB = 2 * IB
NGRP = NSUBB // 4
NBATCH = CPT // IB
ROWS_PT = N // NSUB
ZROWS = 125

RB = 1000
NBLK = N // RB


def _onehot_f32(b_idx):
    iota = lax.broadcasted_iota(jnp.int32, (RB, G), 1)
    return jnp.where(b_idx[:, None] == iota, 1.0, 0.0).astype(jnp.float32)


def _proj_body(x_ref, b_ref, cond_ref, gf_ref, w1a_ref, p1_ref):
    cg = jnp.concatenate([cond_ref[...], gf_ref[...]], axis=1)
    gp1 = jnp.dot(cg, w1a_ref[D:], preferred_element_type=jnp.float32)
    oh = _onehot_f32(b_ref[0, 0, :])
    p1_ref[...] = (
        jnp.dot(x_ref[...], w1a_ref[:D], preferred_element_type=jnp.float32)
        + jnp.dot(oh, gp1, preferred_element_type=jnp.float32)
    )


def _mid_body(p_ref, a_ref, b_ref, cond_ref, gf_ref, ba_ref, wb_ref, bb_ref,
              w2a_ref, o_ref):
    z = p_ref[...] + a_ref[0] + a_ref[1] + ba_ref[...][None, :]
    t = jnp.where(z >= 0, z, 0.01 * z)
    x1 = jnp.dot(t, wb_ref[...], preferred_element_type=jnp.float32) + bb_ref[...][None, :]
    cg = jnp.concatenate([cond_ref[...], gf_ref[...]], axis=1)
    gp2 = jnp.dot(cg, w2a_ref[D:], preferred_element_type=jnp.float32)
    oh = _onehot_f32(b_ref[0, 0, :])
    o_ref[...] = (
        jnp.dot(x1, w2a_ref[:D], preferred_element_type=jnp.float32)
        + jnp.dot(oh, gp2, preferred_element_type=jnp.float32)
    )


def _final_body(p_ref, a_ref, ba_ref, wb_ref, bb_ref, o_ref):
    z = p_ref[...] + a_ref[0] + a_ref[1] + ba_ref[...][None, :]
    t = jnp.where(z >= 0, z, 0.01 * z)
    o_ref[...] = (jnp.dot(t, wb_ref[...], preferred_element_type=jnp.float32)
                  + bb_ref[...][None, :])


_row_spec = pl.BlockSpec((RB, D), lambda i: (i, 0))
_batch_spec = pl.BlockSpec((1, 1, RB), lambda i: (i, 0, 0))
_agg_spec = pl.BlockSpec((NCORE, RB, D), lambda i: (0, i, 0))


def _full_spec(r, c):
    return pl.BlockSpec((r, c), lambda i: (0, 0))


def _vec_spec():
    return pl.BlockSpec((D,), lambda i: (0,))


_nd_f32 = jax.ShapeDtypeStruct((N, D), jnp.float32)

_proj_call = pl.pallas_call(
    _proj_body,
    grid=(NBLK,),
    in_specs=[_row_spec, _batch_spec, _full_spec(G, 16), _full_spec(G, 16),
              _full_spec(D + CG, D)],
    out_specs=_row_spec,
    out_shape=_nd_f32,
)

_mid_call = pl.pallas_call(
    _mid_body,
    grid=(NBLK,),
    in_specs=[_row_spec, _agg_spec, _batch_spec, _full_spec(G, 16),
              _full_spec(G, 16), _vec_spec(), _full_spec(D, D), _vec_spec(),
              _full_spec(D + CG, D)],
    out_specs=_row_spec,
    out_shape=_nd_f32,
)

_final_call = pl.pallas_call(
    _final_body,
    grid=(NBLK,),
    in_specs=[_row_spec, _agg_spec, _vec_spec(), _full_spec(D, D),
              _vec_spec()],
    out_specs=_row_spec,
    out_shape=_nd_f32,
)


@functools.cache
def _make_segsum_sc():
  mesh = plsc.VectorSubcoreMesh(core_axis_name="c", subcore_axis_name="s")

  @functools.partial(
      pl.kernel,
      out_type=jax.ShapeDtypeStruct((NCORE, N, D), jnp.float32),
      mesh=mesh,
      compiler_params=pltpu.CompilerParams(use_tc_tiling_on_sc=False,
                                           disable_bounds_checks=True),
      scratch_types=[
          pltpu.VMEM((IB, SUB), jnp.int32),
          pltpu.VMEM((IB, SUB), jnp.int32),
          pltpu.VMEM((IB, SUB), jnp.int32),
          pltpu.VMEM((IB, SUB), jnp.int32),
          pltpu.VMEM((SUB, D), jnp.float32),
          pltpu.VMEM((SUB, D), jnp.float32),
          pltpu.VMEM((SUB, D), jnp.float32),
          pltpu.VMEM((SUB, D), jnp.float32),
          pltpu.SemaphoreType.DMA,
          pltpu.SemaphoreType.DMA,
          pltpu.SemaphoreType.DMA,
          pltpu.SemaphoreType.DMA,
          pltpu.SemaphoreType.DMA,
          pltpu.SemaphoreType.DMA,
          pltpu.SemaphoreType.DMA,
          pltpu.SemaphoreType.DMA,
          pltpu.VMEM_SHARED((N, D), jnp.float32),
      ],
  )
  def _segsum_sc(p_hbm, edges_hbm, out_hbm,
                 srcA, srcB, dstA, dstB, r0, r1, r2, r3,
                 g0, g1, g2, g3, s0, s1, s2, s3, acc):
    c = lax.axis_index("c")
    s = lax.axis_index("s")
    wid = c * NSUB + s
    tchunk0 = wid * CPT

    rows = (r0, r1, r2, r3)
    gsem = (g0, g1, g2, g3)
    ssem = (s0, s1, s2, s3)
    srcs = (srcA, srcB)
    dsts = (dstA, dstB)

    def _load_idx(bchunk0, sync):
      cps = [
          (edges_hbm.at[0, pl.ds(bchunk0, IB), pl.ds(0, SUB)], srcA, g0),
          (edges_hbm.at[0, pl.ds(bchunk0, IB), pl.ds(SUB, SUB)], srcB, g1),
          (edges_hbm.at[1, pl.ds(bchunk0, IB), pl.ds(0, SUB)], dstA, g2),
          (edges_hbm.at[1, pl.ds(bchunk0, IB), pl.ds(SUB, SUB)], dstB, g3),
      ]
      if sync:
        for src, dst, _ in cps:
          pltpu.sync_copy(src, dst)
      else:
        for src, dst, sm in cps:
          pltpu.async_copy(src, dst, sm)
        return [pltpu.make_async_copy(src, dst, sm) for src, dst, sm in cps]

    idx_cps = _load_idx(tchunk0, sync=False)

    def _zrow(i, carry):
      for j in range(D // 16):
        r0[i, pl.ds(j * 16, 16)] = jnp.zeros((16,), jnp.float32)
      return carry

    lax.fori_loop(0, SUB, _zrow, 0)
    zdone = 0
    while zdone < ROWS_PT:
      nz = min(SUB, ROWS_PT - zdone)
      pltpu.sync_copy(r0.at[pl.ds(0, nz)],
                      acc.at[pl.ds(s * ROWS_PT + zdone, nz)])
      zdone += nz
    for cp in idx_cps:
      cp.wait()
    plsc.subcore_barrier()

    def _batch(ib, carry):
      bchunk0 = tchunk0 + ib * IB

      @pl.when(ib > 0)
      def _reload():
        _load_idx(bchunk0, sync=True)

      pltpu.async_copy(p_hbm.at[srcA.at[0]], r0, g0)
      pltpu.async_copy(p_hbm.at[srcB.at[0]], r1, g1)

      def _grp(q, carry2):
        for b in range(4):
          h = b % 2
          r = 2 * q + b // 2
          pltpu.make_async_copy(p_hbm.at[srcs[h].at[r]], rows[b], gsem[b]).wait()
          pltpu.async_copy(rows[b], acc.at[dsts[h].at[r]], ssem[b], add=True)

          b2 = (b - 2) % 4
          r2_ = r - 1

          def _rot():
            pltpu.make_async_copy(rows[b2], acc.at[dsts[h].at[r2_]],
                                  ssem[b2]).wait()
            pltpu.async_copy(p_hbm.at[srcs[h].at[r + 1]], rows[b2], gsem[b2])

          if b < 2:
            @pl.when(q > 0)
            def _():
              _rot()

            @pl.when(q == 0)
            def _():
              pltpu.async_copy(p_hbm.at[srcs[h].at[r + 1]], rows[b2], gsem[b2])
          else:
            @pl.when(q < NGRP - 1)
            def _():
              _rot()

            @pl.when(q == NGRP - 1)
            def _():
              pltpu.make_async_copy(rows[b2], acc.at[dsts[h].at[r2_]],
                                    ssem[b2]).wait()
        return carry2

      lax.fori_loop(0, NGRP, _grp, 0)
      pltpu.make_async_copy(r2, acc.at[dstA.at[IB - 1]], s2).wait()
      pltpu.make_async_copy(r3, acc.at[dstB.at[IB - 1]], s3).wait()
      return carry

    lax.fori_loop(0, NBATCH, _batch, 0)

    @pl.when(wid < NEXTRA)
    def _epilogue():
      kx = CPT * NWORK + wid
      pltpu.sync_copy(edges_hbm.at[0, pl.ds(kx, 1), pl.ds(0, SUB)],
                      srcA.at[pl.ds(0, 1)])
      pltpu.sync_copy(edges_hbm.at[0, pl.ds(kx, 1), pl.ds(SUB, SUB)],
                      srcB.at[pl.ds(0, 1)])
      pltpu.sync_copy(edges_hbm.at[1, pl.ds(kx, 1), pl.ds(0, SUB)],
                      dstA.at[pl.ds(0, 1)])
      pltpu.sync_copy(edges_hbm.at[1, pl.ds(kx, 1), pl.ds(SUB, SUB)],
                      dstB.at[pl.ds(0, 1)])
      pltpu.async_copy(p_hbm.at[srcA.at[0]], r0, g0).wait()
      pltpu.sync_copy(r0, acc.at[dstA.at[0]], add=True)
      pltpu.async_copy(p_hbm.at[srcB.at[0]], r1, g1).wait()
      pltpu.sync_copy(r1, acc.at[dstB.at[0]], add=True)

    plsc.subcore_barrier()

    pltpu.sync_copy(acc.at[pl.ds(s * ROWS_PT, ROWS_PT)],
                    out_hbm.at[c, pl.ds(s * ROWS_PT, ROWS_PT)])

  return _segsum_sc


def kernel(x, cond, edge_index, batch, global_features,
           W1a, b1a, W1b, b1b, W2a, b2a, W2b, b2b):
    edges = edge_index.reshape(2, NCHUNK_TOT, CH)
    batch3 = batch.reshape(NBLK, 1, RB)

    segsum_sc = _make_segsum_sc()
    p1 = _proj_call(x, batch3, cond, global_features, W1a)
    agg1 = segsum_sc(p1, edges)
    p2 = _mid_call(p1, agg1, batch3, cond, global_features, b1a, W1b, b1b, W2a)
    agg2 = segsum_sc(p2, edges)
    return _final_call(p2, agg2, b2a, W2b, b2b)


# speedup vs baseline: 1.0914x; 1.0914x over previous
import functools
import jax
import jax.numpy as jnp
from jax import lax
from jax.experimental import pallas as pl
from jax.experimental.pallas import tpu as pltpu
from jax.experimental.pallas import tpu_sc as plsc

N = 10000
E = 320000
D = 128
G = 64
CG = 32
NCORE = 2
NSUB = 16
NWORK = NCORE * NSUB
CH = 128
SUB = 64
NCHUNK_TOT = E // CH
CPT = 78
NEXTRA = NCHUNK_TOT - CPT * NWORK
IB = 26
NSUBB = 2 * IB
NGRP = NSUBB // 4
NBATCH = CPT // IB
ROWS_PT = N // NSUB
ZROWS = 125

RB = 1000
NBLK = N // RB


def _onehot_f32(b_idx):
    iota = lax.broadcasted_iota(jnp.int32, (RB, G), 1)
    return jnp.where(b_idx[:, None] == iota, 1.0, 0.0).astype(jnp.float32)


def _proj_body(x_ref, b_ref, cond_ref, gf_ref, w1a_ref, p1_ref):
    cg = jnp.concatenate([cond_ref[...], gf_ref[...]], axis=1)
    gp1 = jnp.dot(cg, w1a_ref[D:], preferred_element_type=jnp.float32)
    oh = _onehot_f32(b_ref[0, 0, :])
    p1_ref[...] = (
        jnp.dot(x_ref[...], w1a_ref[:D], preferred_element_type=jnp.float32)
        + jnp.dot(oh, gp1, preferred_element_type=jnp.float32)
    )


def _mid_body(p_ref, a_ref, b_ref, cond_ref, gf_ref, ba_ref, wb_ref, bb_ref,
              w2a_ref, o_ref):
    z = p_ref[...] + a_ref[0] + a_ref[1] + ba_ref[...][None, :]
    t = jnp.where(z >= 0, z, 0.01 * z)
    x1 = jnp.dot(t, wb_ref[...], preferred_element_type=jnp.float32) + bb_ref[...][None, :]
    cg = jnp.concatenate([cond_ref[...], gf_ref[...]], axis=1)
    gp2 = jnp.dot(cg, w2a_ref[D:], preferred_element_type=jnp.float32)
    oh = _onehot_f32(b_ref[0, 0, :])
    o_ref[...] = (
        jnp.dot(x1, w2a_ref[:D], preferred_element_type=jnp.float32)
        + jnp.dot(oh, gp2, preferred_element_type=jnp.float32)
    )


def _final_body(p_ref, a_ref, ba_ref, wb_ref, bb_ref, o_ref):
    z = p_ref[...] + a_ref[0] + a_ref[1] + ba_ref[...][None, :]
    t = jnp.where(z >= 0, z, 0.01 * z)
    o_ref[...] = (jnp.dot(t, wb_ref[...], preferred_element_type=jnp.float32)
                  + bb_ref[...][None, :])


_row_spec = pl.BlockSpec((RB, D), lambda i: (i, 0))
_batch_spec = pl.BlockSpec((1, 1, RB), lambda i: (i, 0, 0))
_agg_spec = pl.BlockSpec((NCORE, RB, D), lambda i: (0, i, 0))


def _full_spec(r, c):
    return pl.BlockSpec((r, c), lambda i: (0, 0))


def _vec_spec():
    return pl.BlockSpec((D,), lambda i: (0,))


_nd_f32 = jax.ShapeDtypeStruct((N, D), jnp.float32)

_proj_call = pl.pallas_call(
    _proj_body,
    grid=(NBLK,),
    in_specs=[_row_spec, _batch_spec, _full_spec(G, 16), _full_spec(G, 16),
              _full_spec(D + CG, D)],
    out_specs=_row_spec,
    out_shape=_nd_f32,
)

_mid_call = pl.pallas_call(
    _mid_body,
    grid=(NBLK,),
    in_specs=[_row_spec, _agg_spec, _batch_spec, _full_spec(G, 16),
              _full_spec(G, 16), _vec_spec(), _full_spec(D, D), _vec_spec(),
              _full_spec(D + CG, D)],
    out_specs=_row_spec,
    out_shape=_nd_f32,
)

_final_call = pl.pallas_call(
    _final_body,
    grid=(NBLK,),
    in_specs=[_row_spec, _agg_spec, _vec_spec(), _full_spec(D, D),
              _vec_spec()],
    out_specs=_row_spec,
    out_shape=_nd_f32,
)


@functools.cache
def _make_segsum_sc():
  mesh = plsc.VectorSubcoreMesh(core_axis_name="c", subcore_axis_name="s")

  @functools.partial(
      pl.kernel,
      out_type=jax.ShapeDtypeStruct((NCORE, N, D), jnp.float32),
      mesh=mesh,
      compiler_params=pltpu.CompilerParams(use_tc_tiling_on_sc=False,
                                           disable_bounds_checks=True),
      scratch_types=[
          pltpu.VMEM((IB, SUB), jnp.int32),
          pltpu.VMEM((IB, SUB), jnp.int32),
          pltpu.VMEM((IB, SUB), jnp.int32),
          pltpu.VMEM((IB, SUB), jnp.int32),
          pltpu.VMEM((SUB, D), jnp.float32),
          pltpu.VMEM((SUB, D), jnp.float32),
          pltpu.VMEM((SUB, D), jnp.float32),
          pltpu.VMEM((SUB, D), jnp.float32),
          pltpu.SemaphoreType.DMA,
          pltpu.SemaphoreType.DMA,
          pltpu.SemaphoreType.DMA,
          pltpu.SemaphoreType.DMA,
          pltpu.VMEM_SHARED((N, D), jnp.float32),
      ],
  )
  def _segsum_sc(p_hbm, edges_hbm, out_hbm,
                 srcA, srcB, dstA, dstB, r0, r1, r2, r3,
                 g0, g1, g2, g3, acc):
    c = lax.axis_index("c")
    s = lax.axis_index("s")
    wid = c * NSUB + s
    tchunk0 = wid * CPT

    rows = (r0, r1, r2, r3)
    gsem = (g0, g1, g2, g3)
    srcs = (srcA, srcB)
    dsts = (dstA, dstB)

    def _load_idx(bchunk0, sync):
      cps = [
          (edges_hbm.at[0, pl.ds(bchunk0, IB), pl.ds(0, SUB)], srcA, g0),
          (edges_hbm.at[0, pl.ds(bchunk0, IB), pl.ds(SUB, SUB)], srcB, g1),
          (edges_hbm.at[1, pl.ds(bchunk0, IB), pl.ds(0, SUB)], dstA, g2),
          (edges_hbm.at[1, pl.ds(bchunk0, IB), pl.ds(SUB, SUB)], dstB, g3),
      ]
      if sync:
        for src, dst, _ in cps:
          pltpu.sync_copy(src, dst)
      else:
        for src, dst, sm in cps:
          pltpu.async_copy(src, dst, sm)
        return [pltpu.make_async_copy(src, dst, sm) for src, dst, sm in cps]

    idx_cps = _load_idx(tchunk0, sync=False)

    def _zrow(i, carry):
      for j in range(D // 16):
        r0[i, pl.ds(j * 16, 16)] = jnp.zeros((16,), jnp.float32)
      return carry

    lax.fori_loop(0, SUB, _zrow, 0)
    zdone = 0
    while zdone < ROWS_PT:
      nz = min(SUB, ROWS_PT - zdone)
      pltpu.sync_copy(r0.at[pl.ds(0, nz)],
                      acc.at[pl.ds(s * ROWS_PT + zdone, nz)])
      zdone += nz
    for cp in idx_cps:
      cp.wait()
    plsc.subcore_barrier()

    def _batch(ib, carry):
      bchunk0 = tchunk0 + ib * IB

      @pl.when(ib > 0)
      def _reload():
        _load_idx(bchunk0, sync=True)

      pltpu.async_copy(p_hbm.at[srcA.at[0]], r0, g0)
      pltpu.async_copy(p_hbm.at[srcB.at[0]], r1, g1)

      def _grp(q, carry2):
        for b in range(4):
          h = b % 2
          r = 2 * q + b // 2
          pltpu.make_async_copy(p_hbm.at[srcs[h].at[r]], rows[b], gsem[b]).wait()
          b2 = (b + 2) % 4
          if b < 2:
            pltpu.async_copy(p_hbm.at[srcs[h].at[r + 1]], rows[b2], gsem[b2])
          else:
            @pl.when(q < NGRP - 1)
            def _():
              pltpu.async_copy(p_hbm.at[srcs[h].at[r + 1]], rows[b2], gsem[b2])
          pltpu.sync_copy(rows[b], acc.at[dsts[h].at[r]], add=True)
        return carry2

      lax.fori_loop(0, NGRP, _grp, 0)
      return carry

    lax.fori_loop(0, NBATCH, _batch, 0)

    @pl.when(wid < NEXTRA)
    def _epilogue():
      kx = CPT * NWORK + wid
      pltpu.sync_copy(edges_hbm.at[0, pl.ds(kx, 1), pl.ds(0, SUB)],
                      srcA.at[pl.ds(0, 1)])
      pltpu.sync_copy(edges_hbm.at[0, pl.ds(kx, 1), pl.ds(SUB, SUB)],
                      srcB.at[pl.ds(0, 1)])
      pltpu.sync_copy(edges_hbm.at[1, pl.ds(kx, 1), pl.ds(0, SUB)],
                      dstA.at[pl.ds(0, 1)])
      pltpu.sync_copy(edges_hbm.at[1, pl.ds(kx, 1), pl.ds(SUB, SUB)],
                      dstB.at[pl.ds(0, 1)])
      pltpu.async_copy(p_hbm.at[srcA.at[0]], r0, g0).wait()
      pltpu.sync_copy(r0, acc.at[dstA.at[0]], add=True)
      pltpu.async_copy(p_hbm.at[srcB.at[0]], r1, g1).wait()
      pltpu.sync_copy(r1, acc.at[dstB.at[0]], add=True)

    plsc.subcore_barrier()

    pltpu.sync_copy(acc.at[pl.ds(s * ROWS_PT, ROWS_PT)],
                    out_hbm.at[c, pl.ds(s * ROWS_PT, ROWS_PT)])

  return _segsum_sc


def kernel(x, cond, edge_index, batch, global_features,
           W1a, b1a, W1b, b1b, W2a, b2a, W2b, b2b):
    edges = edge_index.reshape(2, NCHUNK_TOT, CH)
    batch3 = batch.reshape(NBLK, 1, RB)

    segsum_sc = _make_segsum_sc()
    p1 = _proj_call(x, batch3, cond, global_features, W1a)
    agg1 = segsum_sc(p1, edges)
    p2 = _mid_call(p1, agg1, batch3, cond, global_features, b1a, W1b, b1b, W2a)
    agg2 = segsum_sc(p2, edges)
    return _final_call(p2, agg2, b2a, W2b, b2b)
